# baseline (speedup 1.0000x reference)
import jax
import jax.numpy as jnp
from jax.experimental import pallas as pl
from jax.experimental.pallas import tpu as pltpu

B, H, S, D = 8, 4, 2048, 128
SW = 512
TS = 1024
TPS = S // TS


def _body(pos_ref, k0, v0, k1, v1, lat, out):
    out[0] = k0[...]
    out[1] = v0[...]
    out[2] = k1[...]
    out[3] = v1[...]

    t = pl.program_id(0)
    bh = t // TPS
    s_lo = (t % TPS) * TS
    b = bh // H
    p0 = pos_ref[b]
    p1 = jax.lax.rem(p0, SW)

    r0 = p0 - s_lo

    @pl.when((r0 >= 0) & (r0 < TS))
    def _():
        out[0, pl.ds(r0, 1), :] = lat[0, 0]
        out[1, pl.ds(r0, 1), :] = lat[0, 1]

    r1 = p1 - s_lo

    @pl.when((r1 >= 0) & (r1 < TS))
    def _():
        out[2, pl.ds(r1, 1), :] = lat[0, 2]
        out[3, pl.ds(r1, 1), :] = lat[0, 3]


def kernel(k_cache_0, v_cache_0, k_cache_1, v_cache_1,
           latest_k_0, latest_v_0, latest_k_1, latest_v_1, position_ids):
    caches = [c.reshape(B * H * S, D)
              for c in (k_cache_0, v_cache_0, k_cache_1, v_cache_1)]
    lat = jnp.stack([latest_k_0, latest_v_0, latest_k_1, latest_v_1],
                    axis=3).reshape(B * H, 4, 1, D)
    pos = position_ids.reshape(B).astype(jnp.int32)

    n_steps = B * H * TPS
    grid_spec = pltpu.PrefetchScalarGridSpec(
        num_scalar_prefetch=1,
        grid=(n_steps,),
        in_specs=[
            pl.BlockSpec((TS, D), lambda t, pref: (t, 0)),
            pl.BlockSpec((TS, D), lambda t, pref: (t, 0)),
            pl.BlockSpec((TS, D), lambda t, pref: (t, 0)),
            pl.BlockSpec((TS, D), lambda t, pref: (t, 0)),
            pl.BlockSpec((1, 4, 1, D), lambda t, pref: (t // TPS, 0, 0, 0)),
        ],
        out_specs=pl.BlockSpec((4, TS, D), lambda t, pref: (0, t, 0)),
    )
    out = pl.pallas_call(
        _body,
        grid_spec=grid_spec,
        out_shape=jax.ShapeDtypeStruct((4, B * H * S, D), jnp.float32),
    )(pos, *caches, lat)
    return out.reshape(4, B, H, S, D)


# speedup vs baseline: 1.1000x; 1.1000x over previous
import jax
import jax.numpy as jnp
from jax.experimental import pallas as pl
from jax.experimental.pallas import tpu as pltpu

B, H, S, D = 8, 4, 2048, 128
SW = 512
SLABS = 2
TS = SLABS * S


def _body(pos_ref, k0, v0, k1, v1, lat, out):
    out[0] = k0[...]
    out[1] = v0[...]
    out[2] = k1[...]
    out[3] = v1[...]

    t = pl.program_id(0)
    for j in range(SLABS):
        bh = t * SLABS + j
        b = bh // H
        p0 = pos_ref[b]
        p1 = jax.lax.rem(p0, SW)
        out[0, pl.ds(j * S + p0, 1), :] = lat[j, 0]
        out[1, pl.ds(j * S + p0, 1), :] = lat[j, 1]
        out[2, pl.ds(j * S + p1, 1), :] = lat[j, 2]
        out[3, pl.ds(j * S + p1, 1), :] = lat[j, 3]


def kernel(k_cache_0, v_cache_0, k_cache_1, v_cache_1,
           latest_k_0, latest_v_0, latest_k_1, latest_v_1, position_ids):
    caches = [c.reshape(B * H * S, D)
              for c in (k_cache_0, v_cache_0, k_cache_1, v_cache_1)]
    lat = jnp.stack([latest_k_0, latest_v_0, latest_k_1, latest_v_1],
                    axis=3).reshape(B * H, 4, 1, D)
    pos = position_ids.reshape(B).astype(jnp.int32)

    n_steps = B * H // SLABS
    grid_spec = pltpu.PrefetchScalarGridSpec(
        num_scalar_prefetch=1,
        grid=(n_steps,),
        in_specs=[
            pl.BlockSpec((TS, D), lambda t, pref: (t, 0)),
            pl.BlockSpec((TS, D), lambda t, pref: (t, 0)),
            pl.BlockSpec((TS, D), lambda t, pref: (t, 0)),
            pl.BlockSpec((TS, D), lambda t, pref: (t, 0)),
            pl.BlockSpec((SLABS, 4, 1, D), lambda t, pref: (t, 0, 0, 0)),
        ],
        out_specs=pl.BlockSpec((4, TS, D), lambda t, pref: (0, t, 0)),
    )
    out = pl.pallas_call(
        _body,
        grid_spec=grid_spec,
        out_shape=jax.ShapeDtypeStruct((4, B * H * S, D), jnp.float32),
    )(pos, *caches, lat)
    return out.reshape(4, B, H, S, D)
